# baseline (speedup 1.0000x reference)
import jax
import jax.numpy as jnp
from jax import lax
from jax.experimental import pallas as pl


_K = 9
_TR = 128


def _body(xrow_ref, xall_ref, wbd_ref, b_ref, out_ref):
    xrow = xrow_ref[0]
    xall = xall_ref[0]

    nall = jnp.sqrt(jnp.sum(xall * xall, axis=1, keepdims=True))
    xn_all = xall / jnp.maximum(nall, 1e-12)
    nrow = jnp.sqrt(jnp.sum(xrow * xrow, axis=1, keepdims=True))
    xn_row = xrow / jnp.maximum(nrow, 1e-12)
    inner = lax.dot_general(
        xn_row, xn_all, (((1,), (1,)), ((), ())),
        preferred_element_type=jnp.float32,
    )
    sq_all = jnp.sum(xn_all * xn_all, axis=1)
    sq_row = jnp.sum(xn_row * xn_row, axis=1)
    neg = -((sq_row[:, None] + (-2.0) * inner) + sq_all[None, :])

    zrow = lax.dot_general(
        xrow, wbd_ref[...], (((1,), (1,)), ((), ())),
        preferred_element_type=jnp.float32,
    )
    y_all = lax.dot_general(
        xall, wbd_ref[pl.ds(48, 48), :], (((1,), (1,)), ((), ())),
        preferred_element_type=jnp.float32,
    )
    bvec = b_ref[0]
    out1 = jnp.maximum(zrow[:, :48] + bvec[None, :48], 0.0)
    y_row = zrow[:, 48:]
    b2 = bvec[None, 48:]

    acc = jnp.zeros_like(y_row)
    for _ in range(_K):
        m = jnp.max(neg, axis=1, keepdims=True)
        onehot = (neg == m).astype(jnp.float32)
        neg = neg - onehot * 3.0e38
        yj = lax.dot_general(
            onehot, y_all, (((1,), (0,)), ((), ())),
            preferred_element_type=jnp.float32,
        )
        acc += jnp.maximum(yj - y_row + b2, 0.0)

    out_ref[0] = jnp.concatenate([out1, acc * (1.0 / _K)], axis=1)


def kernel(x, W, b):
    Bb, Cc, Hh, Ww = x.shape
    N = Hh * Ww
    Cout = W.shape[0]
    half = Cc // 2

    xt = jnp.transpose(x.reshape(Bb, Cc, N), (0, 2, 1))

    Wbd = jnp.zeros((Cout, Cc), dtype=W.dtype)
    Wbd = Wbd.at[0:24, 0:half].set(W[0:24])
    Wbd = Wbd.at[24:48, half:Cc].set(W[24:48])
    Wbd = Wbd.at[48:72, 0:half].set(W[48:72])
    Wbd = Wbd.at[72:96, half:Cc].set(W[72:96])

    nt = N // _TR
    out = pl.pallas_call(
        _body,
        grid=(Bb, nt),
        in_specs=[
            pl.BlockSpec((1, _TR, Cc), lambda i, r: (i, r, 0)),
            pl.BlockSpec((1, N, Cc), lambda i, r: (i, 0, 0)),
            pl.BlockSpec((Cout, Cc), lambda i, r: (0, 0)),
            pl.BlockSpec((1, Cout), lambda i, r: (0, 0)),
        ],
        out_specs=pl.BlockSpec((1, _TR, Cout), lambda i, r: (i, r, 0)),
        out_shape=jax.ShapeDtypeStruct((Bb, N, Cout), jnp.float32),
    )(xt, xt, Wbd, b.reshape(1, Cout))

    return jnp.transpose(out, (0, 2, 1)).reshape(Bb, Cout, Hh, Ww)


# speedup vs baseline: 1.2466x; 1.2466x over previous
import jax
import jax.numpy as jnp
from jax import lax
from jax.experimental import pallas as pl


_K = 9
_TR = 512


def _body(xrow_ref, xall_ref, wbd_ref, b_ref, out_ref):
    xrow = xrow_ref[0]
    xall = xall_ref[0]

    nall = jnp.sqrt(jnp.sum(xall * xall, axis=1, keepdims=True))
    xn_all = xall / jnp.maximum(nall, 1e-12)
    nrow = jnp.sqrt(jnp.sum(xrow * xrow, axis=1, keepdims=True))
    xn_row = xrow / jnp.maximum(nrow, 1e-12)
    inner = lax.dot_general(
        xn_row, xn_all, (((1,), (1,)), ((), ())),
        preferred_element_type=jnp.float32,
    )
    sq_all = jnp.sum(xn_all * xn_all, axis=1)
    sq_row = jnp.sum(xn_row * xn_row, axis=1)
    neg = -((sq_row[:, None] + (-2.0) * inner) + sq_all[None, :])

    zrow = lax.dot_general(
        xrow, wbd_ref[...], (((1,), (1,)), ((), ())),
        preferred_element_type=jnp.float32,
    )
    y_all = lax.dot_general(
        xall, wbd_ref[pl.ds(48, 48), :], (((1,), (1,)), ((), ())),
        preferred_element_type=jnp.float32,
    )
    bvec = b_ref[0]
    out1 = jnp.maximum(zrow[:, :48] + bvec[None, :48], 0.0)
    y_row = zrow[:, 48:]
    b2 = bvec[None, 48:]

    acc = jnp.zeros_like(y_row)
    for _ in range(_K):
        m = jnp.max(neg, axis=1, keepdims=True)
        onehot = (neg == m).astype(jnp.float32)
        neg = neg - onehot * 3.0e38
        yj = lax.dot_general(
            onehot, y_all, (((1,), (0,)), ((), ())),
            preferred_element_type=jnp.float32,
        )
        acc += jnp.maximum(yj - y_row + b2, 0.0)

    out_ref[0] = jnp.concatenate([out1, acc * (1.0 / _K)], axis=1)


def kernel(x, W, b):
    Bb, Cc, Hh, Ww = x.shape
    N = Hh * Ww
    Cout = W.shape[0]
    half = Cc // 2

    xt = jnp.transpose(x.reshape(Bb, Cc, N), (0, 2, 1))

    Wbd = jnp.zeros((Cout, Cc), dtype=W.dtype)
    Wbd = Wbd.at[0:24, 0:half].set(W[0:24])
    Wbd = Wbd.at[24:48, half:Cc].set(W[24:48])
    Wbd = Wbd.at[48:72, 0:half].set(W[48:72])
    Wbd = Wbd.at[72:96, half:Cc].set(W[72:96])

    nt = N // _TR
    out = pl.pallas_call(
        _body,
        grid=(Bb, nt),
        in_specs=[
            pl.BlockSpec((1, _TR, Cc), lambda i, r: (i, r, 0)),
            pl.BlockSpec((1, N, Cc), lambda i, r: (i, 0, 0)),
            pl.BlockSpec((Cout, Cc), lambda i, r: (0, 0)),
            pl.BlockSpec((1, Cout), lambda i, r: (0, 0)),
        ],
        out_specs=pl.BlockSpec((1, _TR, Cout), lambda i, r: (i, r, 0)),
        out_shape=jax.ShapeDtypeStruct((Bb, N, Cout), jnp.float32),
    )(xt, xt, Wbd, b.reshape(1, Cout))

    return jnp.transpose(out, (0, 2, 1)).reshape(Bb, Cout, Hh, Ww)
